# baseline (speedup 1.0000x reference)
import dataclasses
import functools

import jax
import jax.numpy as jnp
from jax import lax
from jax.experimental import pallas as pl
from jax.experimental.pallas import tpu as pltpu
from jax.experimental.pallas import tpu_sc as plsc

N = 10000
E = 320000
D = 128
NC = 2
NS = 16
NW = NC * NS
CH = 128
NCH = 80
EPP = NCH * CH
EP = NW * EPP
NBUF = 2
NQ = 4
CPQ = NCH // NQ
EPQ = CPQ * CH
NP = 10240
RPS = NP // NS

BM = 1024
GRID = NP // BM

_mesh = plsc.VectorSubcoreMesh(core_axis_name="c", subcore_axis_name="s")

_sc_params = pltpu.CompilerParams()
if "needs_layout_passes" in pltpu.CompilerParams.__dataclass_fields__:
    _sc_params = dataclasses.replace(_sc_params, needs_layout_passes=False)


@functools.partial(
    pl.kernel,
    out_type=jax.ShapeDtypeStruct((NW * NP,), jnp.float32),
    mesh=_mesh,
    compiler_params=_sc_params,
    scratch_types=[
        pltpu.VMEM((EPP,), jnp.int32),
        pltpu.VMEM((NP,), jnp.float32),
    ],
)
def _deg_kernel(dst_hbm, z_hbm, out_hbm, dst_all, hist):
    c = lax.axis_index("c")
    s = lax.axis_index("s")
    wid = s * NC + c
    base = wid * EPP
    pltpu.sync_copy(z_hbm, hist)
    pltpu.sync_copy(dst_hbm.at[pl.ds(base, EPP)], dst_all)
    ones = jnp.ones((16,), jnp.float32)

    @pl.loop(0, EPP // 16, step=5)
    def _(k):
        for i in range(5):
            idx = dst_all[pl.ds((k + i) * 16, 16)]
            plsc.addupdate_scatter(hist, [idx], ones)

    pltpu.sync_copy(hist, out_hbm.at[pl.ds(wid * NP, NP)])


@functools.partial(
    pl.kernel,
    out_type=jax.ShapeDtypeStruct((NC, NP, D), jnp.float32),
    mesh=_mesh,
    scratch_types=[
        pltpu.VMEM((EPQ,), jnp.int32),
        pltpu.VMEM((EPQ,), jnp.int32),
        pltpu.VMEM((CPQ, CH), jnp.int32),
        pltpu.VMEM((CPQ, CH), jnp.int32),
        pltpu.VMEM((NBUF, CH, D), jnp.float32),
        pltpu.VMEM_SHARED((NP, D), jnp.float32),
        pltpu.SemaphoreType.DMA,
        pltpu.SemaphoreType.DMA,
        pltpu.SemaphoreType.DMA,
        pltpu.SemaphoreType.DMA,
        pltpu.SemaphoreType.DMA,
        pltpu.SemaphoreType.DMA,
    ],
)
def _edge_agg(g_hbm, src_hbm, dstq_hbm, z_hbm, out_hbm,
              src_q0, src_q1, dst_q0, dst_q1, rows,
              acc, g0, g1_, w0, w1, qs0, qs1):
    src_qs = (src_q0, src_q1)
    dst_qs = (dst_q0, dst_q1)
    gsems = (g0, g1_)
    wsems = (w0, w1)
    qsems = (qs0, qs1)
    c = lax.axis_index("c")
    s = lax.axis_index("s")
    wid = s * NC + c
    base = wid * EPP
    r0 = s * RPS
    pltpu.sync_copy(z_hbm.at[pl.ds(r0, RPS)], acc.at[pl.ds(r0, RPS)])

    def qload(h):
        b = h % 2
        pltpu.async_copy(
            src_hbm.at[pl.ds(base + h * EPQ, EPQ)], src_qs[b], qsems[b])
        pltpu.async_copy(dstq_hbm.at[wid, h], dst_qs[b], qsems[b])

    def qwait(h):
        b = h % 2
        pltpu.make_async_copy(
            src_hbm.at[pl.ds(base + h * EPQ, EPQ)], src_qs[b], qsems[b]).wait()
        pltpu.make_async_copy(
            dstq_hbm.at[wid, h], dst_qs[b], qsems[b]).wait()

    qload(0)
    plsc.subcore_barrier()

    for h in range(NQ):
        src_q, dst_q = src_qs[h % 2], dst_qs[h % 2]

        def startg(jq, k):
            pltpu.async_copy(
                g_hbm.at[src_q.at[pl.ds(jq * CH, CH)]], rows.at[k], gsems[k])

        def waitg(jq, k):
            pltpu.make_async_copy(
                g_hbm.at[src_q.at[pl.ds(jq * CH, CH)]], rows.at[k],
                gsems[k]).wait()

        def startscat(jq, k):
            pltpu.async_copy(
                rows.at[k], acc.at[dst_q.at[jq]], wsems[k], add=True)

        def waitscat(jq, k):
            pltpu.make_async_copy(
                rows.at[k], acc.at[dst_q.at[jq]], wsems[k]).wait()

        qwait(h)
        if h + 1 < NQ:
            qload(h + 1)
        for k in range(NBUF):
            startg(k, k)

        @pl.loop(0, CPQ // NBUF - 1)
        def _(t):
            j0 = NBUF * t
            for k in range(NBUF):
                waitg(j0 + k, k)
                startscat(j0 + k, k)
            for k in range(NBUF):
                waitscat(j0 + k, k)
                startg(j0 + NBUF + k, k)

        j0 = CPQ - NBUF
        for k in range(NBUF):
            waitg(j0 + k, k)
            startscat(j0 + k, k)
        for k in range(NBUF):
            waitscat(j0 + k, k)

    plsc.subcore_barrier()
    pltpu.sync_copy(acc.at[pl.ds(r0, RPS)], out_hbm.at[c, pl.ds(r0, RPS)])


_DN = (((1,), (0,)), ((), ()))


def _dinv_of(deg_blk):
    cnt = jnp.sum(deg_blk, axis=0)
    return lax.rsqrt(cnt + 1.0)


def _lrelu(v):
    return jnp.where(v >= 0, v, 0.01 * v)


def _mmscale_body(deg_ref, x_ref, w_ref, o_ref):
    dinv = _dinv_of(deg_ref[...])
    h = lax.dot_general(
        x_ref[...], w_ref[...], _DN,
        preferred_element_type=jnp.float32, precision=lax.Precision.HIGHEST)
    o_ref[...] = h * dinv[:, None]


def _mid_body(deg_ref, sp_ref, g_ref, b_ref, w_ref, o_ref):
    dinv = _dinv_of(deg_ref[...])
    sp = sp_ref[...]
    h = (sp[0] + sp[1] + g_ref[...]) * dinv[:, None] + b_ref[...]
    h = _lrelu(h)
    o_ref[...] = lax.dot_general(
        h, w_ref[...], _DN,
        preferred_element_type=jnp.float32,
        precision=lax.Precision.HIGHEST) * dinv[:, None]


def _final_body(deg_ref, sp_ref, g_ref, b_ref, o_ref):
    dinv = _dinv_of(deg_ref[...])
    sp = sp_ref[...]
    h = (sp[0] + sp[1] + g_ref[...]) * dinv[:, None] + b_ref[...]
    o_ref[...] = _lrelu(h)


def _row_spec(w):
    return pl.BlockSpec((BM, w), lambda i: (i, 0))


_deg_spec = pl.BlockSpec((NW, BM), lambda i: (0, i))
_sp_spec = pl.BlockSpec((NC, BM, D), lambda i: (0, i, 0))
_w_spec = pl.BlockSpec((D, D), lambda i: (0, 0))
_b_spec = pl.BlockSpec((1, D), lambda i: (0, 0))
_out_f32 = jax.ShapeDtypeStruct((NP, D), jnp.float32)

_mmscale = pl.pallas_call(
    _mmscale_body, grid=(GRID,),
    in_specs=[_deg_spec, _row_spec(D), _w_spec],
    out_specs=_row_spec(D), out_shape=_out_f32)

_mid = pl.pallas_call(
    _mid_body, grid=(GRID,),
    in_specs=[_deg_spec, _sp_spec, _row_spec(D), _b_spec, _w_spec],
    out_specs=_row_spec(D), out_shape=_out_f32)

_final = pl.pallas_call(
    _final_body, grid=(GRID,),
    in_specs=[_deg_spec, _sp_spec, _row_spec(D), _b_spec],
    out_specs=_row_spec(D), out_shape=_out_f32)


def kernel(x, edge_index, W1, b1, W2, b2):
    src = edge_index[0].astype(jnp.int32)
    dst = edge_index[1].astype(jnp.int32)
    zD = jnp.zeros((NP, D), jnp.float32)
    z1 = jnp.zeros((NP,), jnp.float32)
    b1r = b1.reshape(1, D)
    b2r = b2.reshape(1, D)

    xp = jnp.concatenate([x, jnp.zeros((NP - N, D), jnp.float32)], axis=0)

    pad = jnp.arange(EP - E, dtype=jnp.int32)
    srcp = jnp.concatenate([src, pad % N])
    dstp = jnp.concatenate([dst, N + pad % (NP - N)])
    dstq = dstp.reshape(NW, NQ, CPQ, CH)

    deg2 = _deg_kernel(dstp, z1).reshape(NW, NP)
    g1 = _mmscale(deg2, xp, W1)
    S1 = _edge_agg(g1, srcp, dstq, zD)
    g2 = _mid(deg2, S1, g1, b1r, W2)
    S2 = _edge_agg(g2, srcp, dstq, zD)
    out = _final(deg2, S2, g2, b2r)
    return out[:N]


# speedup vs baseline: 1.2315x; 1.2315x over previous
import dataclasses
import functools

import jax
import jax.numpy as jnp
from jax import lax
from jax.experimental import pallas as pl
from jax.experimental.pallas import tpu as pltpu
from jax.experimental.pallas import tpu_sc as plsc

N = 10000
E = 320000
D = 128
NC = 2
NS = 16
NW = NC * NS
CH = 128
NCH = 80
EPP = NCH * CH
EP = NW * EPP
NBUF = 2
NQ = 4
CPQ = NCH // NQ
EPQ = CPQ * CH
NP = 10240
RPS = NP // NS

BM = 1024
GRID = NP // BM

_mesh = plsc.VectorSubcoreMesh(core_axis_name="c", subcore_axis_name="s")

_sc_params = pltpu.CompilerParams()
if "needs_layout_passes" in pltpu.CompilerParams.__dataclass_fields__:
    _sc_params = dataclasses.replace(_sc_params, needs_layout_passes=False)


@functools.partial(
    pl.kernel,
    out_type=jax.ShapeDtypeStruct((NW * NP,), jnp.float32),
    mesh=_mesh,
    compiler_params=_sc_params,
    scratch_types=[
        pltpu.VMEM((EPP,), jnp.int32),
        pltpu.VMEM((NP,), jnp.float32),
    ],
)
def _deg_kernel(dst_hbm, z_hbm, out_hbm, dst_all, hist):
    c = lax.axis_index("c")
    s = lax.axis_index("s")
    wid = s * NC + c
    base = wid * EPP
    pltpu.sync_copy(z_hbm, hist)
    pltpu.sync_copy(dst_hbm.at[pl.ds(base, EPP)], dst_all)
    ones = jnp.ones((16,), jnp.float32)

    @pl.loop(0, EPP // 16, step=5)
    def _(k):
        for i in range(5):
            idx = dst_all[pl.ds((k + i) * 16, 16)]
            plsc.addupdate_scatter(hist, [idx], ones)

    pltpu.sync_copy(hist, out_hbm.at[pl.ds(wid * NP, NP)])


@functools.partial(
    pl.kernel,
    out_type=jax.ShapeDtypeStruct((NC, NP, D), jnp.float32),
    mesh=_mesh,
    scratch_types=[
        pltpu.VMEM((EPQ,), jnp.int32),
        pltpu.VMEM((EPQ,), jnp.int32),
        pltpu.VMEM((CPQ, CH), jnp.int32),
        pltpu.VMEM((CPQ, CH), jnp.int32),
        pltpu.VMEM((NBUF, CH, D), jnp.float32),
        pltpu.VMEM_SHARED((NP, D), jnp.float32),
        pltpu.SemaphoreType.DMA,
        pltpu.SemaphoreType.DMA,
        pltpu.SemaphoreType.DMA,
        pltpu.SemaphoreType.DMA,
        pltpu.SemaphoreType.DMA,
    ],
)
def _edge_agg(g_hbm, src_hbm, dstq_hbm, z_hbm, out_hbm,
              src_q0, src_q1, dst_q0, dst_q1, rows,
              acc, g0, g1_, zs, qs0, qs1):
    src_qs = (src_q0, src_q1)
    dst_qs = (dst_q0, dst_q1)
    gsems = (g0, g1_)
    qsems = (qs0, qs1)
    c = lax.axis_index("c")
    s = lax.axis_index("s")
    wid = s * NC + c
    base = wid * EPP
    r0 = s * RPS
    pltpu.async_copy(z_hbm.at[pl.ds(r0, RPS)], acc.at[pl.ds(r0, RPS)], zs)

    def qload(h):
        b = h % 2
        pltpu.async_copy(
            src_hbm.at[pl.ds(base + h * EPQ, EPQ)], src_qs[b], qsems[b])
        pltpu.async_copy(dstq_hbm.at[wid, h], dst_qs[b], qsems[b])

    def qwait(h):
        b = h % 2
        pltpu.make_async_copy(
            src_hbm.at[pl.ds(base + h * EPQ, EPQ)], src_qs[b], qsems[b]).wait()
        pltpu.make_async_copy(
            dstq_hbm.at[wid, h], dst_qs[b], qsems[b]).wait()

    qload(0)
    pltpu.make_async_copy(
        z_hbm.at[pl.ds(r0, RPS)], acc.at[pl.ds(r0, RPS)], zs).wait()
    plsc.subcore_barrier()

    for h in range(NQ):
        src_q, dst_q = src_qs[h % 2], dst_qs[h % 2]

        def startg(jq, k):
            pltpu.async_copy(
                g_hbm.at[src_q.at[pl.ds(jq * CH, CH)]], rows.at[k], gsems[k])

        def waitg(jq, k):
            pltpu.make_async_copy(
                g_hbm.at[src_q.at[pl.ds(jq * CH, CH)]], rows.at[k],
                gsems[k]).wait()

        def scat(jq, k):
            pltpu.sync_copy(rows.at[k], acc.at[dst_q.at[jq]], add=True)

        qwait(h)
        if h + 1 < NQ:
            qload(h + 1)
        for k in range(NBUF):
            startg(k, k)

        @pl.loop(0, CPQ // NBUF - 1)
        def _(t):
            j0 = NBUF * t
            for k in range(NBUF):
                waitg(j0 + k, k)
                scat(j0 + k, k)
                startg(j0 + NBUF + k, k)

        j0 = CPQ - NBUF
        for k in range(NBUF):
            waitg(j0 + k, k)
            scat(j0 + k, k)

    plsc.subcore_barrier()
    pltpu.sync_copy(acc.at[pl.ds(r0, RPS)], out_hbm.at[c, pl.ds(r0, RPS)])


_DN = (((1,), (0,)), ((), ()))


def _dinv_of(deg_blk):
    cnt = jnp.sum(deg_blk, axis=0)
    return lax.rsqrt(cnt + 1.0)


def _lrelu(v):
    return jnp.where(v >= 0, v, 0.01 * v)


def _mmscale_body(deg_ref, x_ref, w_ref, o_ref):
    dinv = _dinv_of(deg_ref[...])
    h = lax.dot_general(
        x_ref[...], w_ref[...], _DN,
        preferred_element_type=jnp.float32, precision=lax.Precision.HIGHEST)
    o_ref[...] = h * dinv[:, None]


def _mid_body(deg_ref, sp_ref, g_ref, b_ref, w_ref, o_ref):
    dinv = _dinv_of(deg_ref[...])
    sp = sp_ref[...]
    h = (sp[0] + sp[1] + g_ref[...]) * dinv[:, None] + b_ref[...]
    h = _lrelu(h)
    o_ref[...] = lax.dot_general(
        h, w_ref[...], _DN,
        preferred_element_type=jnp.float32,
        precision=lax.Precision.HIGHEST) * dinv[:, None]


def _final_body(deg_ref, sp_ref, g_ref, b_ref, o_ref):
    dinv = _dinv_of(deg_ref[...])
    sp = sp_ref[...]
    h = (sp[0] + sp[1] + g_ref[...]) * dinv[:, None] + b_ref[...]
    o_ref[...] = _lrelu(h)


def _row_spec(w):
    return pl.BlockSpec((BM, w), lambda i: (i, 0))


_deg_spec = pl.BlockSpec((NW, BM), lambda i: (0, i))
_sp_spec = pl.BlockSpec((NC, BM, D), lambda i: (0, i, 0))
_w_spec = pl.BlockSpec((D, D), lambda i: (0, 0))
_b_spec = pl.BlockSpec((1, D), lambda i: (0, 0))
_out_f32 = jax.ShapeDtypeStruct((N, D), jnp.float32)

_mmscale = pl.pallas_call(
    _mmscale_body, grid=(GRID,),
    in_specs=[_deg_spec, _row_spec(D), _w_spec],
    out_specs=_row_spec(D), out_shape=_out_f32)

_mid = pl.pallas_call(
    _mid_body, grid=(GRID,),
    in_specs=[_deg_spec, _sp_spec, _row_spec(D), _b_spec, _w_spec],
    out_specs=_row_spec(D), out_shape=_out_f32)

_final = pl.pallas_call(
    _final_body, grid=(GRID,),
    in_specs=[_deg_spec, _sp_spec, _row_spec(D), _b_spec],
    out_specs=_row_spec(D), out_shape=_out_f32)


def kernel(x, edge_index, W1, b1, W2, b2):
    src = edge_index[0].astype(jnp.int32)
    dst = edge_index[1].astype(jnp.int32)
    zD = jnp.zeros((NP, D), jnp.float32)
    z1 = jnp.zeros((NP,), jnp.float32)
    b1r = b1.reshape(1, D)
    b2r = b2.reshape(1, D)

    pad = jnp.arange(EP - E, dtype=jnp.int32)
    srcp = jnp.concatenate([src, pad % N])
    dstp = jnp.concatenate([dst, N + pad % (NP - N)])
    dstq = dstp.reshape(NW, NQ, CPQ, CH)

    deg2 = _deg_kernel(dstp, z1).reshape(NW, NP)
    g1 = _mmscale(deg2, x, W1)
    S1 = _edge_agg(g1, srcp, dstq, zD)
    g2 = _mid(deg2, S1, g1, b1r, W2)
    S2 = _edge_agg(g2, srcp, dstq, zD)
    return _final(deg2, S2, g2, b2r)
